# Initial kernel scaffold; baseline (speedup 1.0000x reference)
import jax
import jax.numpy as jnp
from jax.experimental import pallas as pl


def kernel(x, pos_table):
    raise NotImplementedError("write your pallas kernel here")


# speedup vs baseline: 3.2925x; 3.2925x over previous
import functools

import jax
import jax.numpy as jnp
from jax.experimental import pallas as pl
from jax.experimental.pallas import tpu as pltpu

BLK = 512


def _add_kernel(x_ref, pos_ref, out_ref):
    out_ref[...] = x_ref[...] + pos_ref[...][None, :, :]


@jax.jit
def kernel(x, pos_table):
    batch, seq_len, emb = x.shape
    n_blk = seq_len // BLK
    return pl.pallas_call(
        _add_kernel,
        grid=(n_blk,),
        in_specs=[
            pl.BlockSpec((batch, BLK, emb), lambda i: (0, i, 0)),
            pl.BlockSpec((BLK, emb), lambda i: (i, 0)),
        ],
        out_specs=pl.BlockSpec((batch, BLK, emb), lambda i: (0, i, 0)),
        out_shape=jax.ShapeDtypeStruct((batch, seq_len, emb), x.dtype),
        compiler_params=pltpu.CompilerParams(
            dimension_semantics=("arbitrary",),
        ),
    )(x, pos_table[:seq_len])
